# baseline (speedup 1.0000x reference)
import functools

import jax
import jax.numpy as jnp
from jax.experimental import pallas as pl
from jax.experimental.pallas import tpu as pltpu

CAPACITY = 100000
INPUT_DIM = 512
VALUE_DIM = 256
THRESHOLD = 0.85 + 0.05
EPS = 1e-12
SCALE = 10.0

BLOCK = 2000
NBLK = CAPACITY // BLOCK


def _body(q_ref, k_ref, v_ref, recall_ref, best_ref, acc_ref, m_ref, l_ref):
    i = pl.program_id(0)

    @pl.when(i == 0)
    def _init():
        m_ref[0, 0] = -jnp.inf
        l_ref[0, 0] = 0.0
        acc_ref[...] = jnp.zeros_like(acc_ref)

    q = q_ref[0, :]
    qn = q / jnp.maximum(jnp.sqrt(jnp.sum(q * q)), EPS)

    s = jax.lax.dot_general(
        qn[None, :], k_ref[...],
        dimension_numbers=(((1,), (1,)), ((), ())),
        preferred_element_type=jnp.float32,
    )

    m_prev = m_ref[0, 0]
    m_new = jnp.maximum(m_prev, jnp.max(s))
    c = jnp.exp(SCALE * (m_prev - m_new))
    p = jnp.exp(SCALE * (s - m_new))
    l_ref[0, 0] = l_ref[0, 0] * c + jnp.sum(p)
    pv = jax.lax.dot_general(
        p, v_ref[...],
        dimension_numbers=(((1,), (0,)), ((), ())),
        preferred_element_type=jnp.float32,
    )
    acc_ref[...] = acc_ref[...] * c + pv
    m_ref[0, 0] = m_new

    @pl.when(i == NBLK - 1)
    def _fin():
        best = m_ref[0, 0]
        r = acc_ref[...] / l_ref[0, 0]
        recall_ref[...] = jnp.where(best >= THRESHOLD, r, jnp.zeros_like(r))
        best_ref[...] = jnp.full((1, 1), best, dtype=jnp.float32)


@jax.jit
def kernel(query_pattern, keys, values):
    q2 = query_pattern.reshape(1, INPUT_DIM)
    recall, best = pl.pallas_call(
        _body,
        grid=(NBLK,),
        in_specs=[
            pl.BlockSpec((1, INPUT_DIM), lambda i: (0, 0)),
            pl.BlockSpec((BLOCK, INPUT_DIM), lambda i: (i, 0)),
            pl.BlockSpec((BLOCK, VALUE_DIM), lambda i: (i, 0)),
        ],
        out_specs=[
            pl.BlockSpec((1, VALUE_DIM), lambda i: (0, 0)),
            pl.BlockSpec((1, 1), lambda i: (0, 0)),
        ],
        out_shape=[
            jax.ShapeDtypeStruct((1, VALUE_DIM), jnp.float32),
            jax.ShapeDtypeStruct((1, 1), jnp.float32),
        ],
        scratch_shapes=[
            pltpu.VMEM((1, VALUE_DIM), jnp.float32),
            pltpu.SMEM((1, 1), jnp.float32),
            pltpu.SMEM((1, 1), jnp.float32),
        ],
        compiler_params=pltpu.CompilerParams(
            dimension_semantics=("arbitrary",),
        ),
    )(q2, keys, values)
    return recall[0], best[0, 0]


# speedup vs baseline: 1.1420x; 1.1420x over previous
import functools

import jax
import jax.numpy as jnp
from jax.experimental import pallas as pl
from jax.experimental.pallas import tpu as pltpu

CAPACITY = 100000
INPUT_DIM = 512
VALUE_DIM = 256
THRESHOLD = 0.85 + 0.05
EPS = 1e-12
SCALE = 10.0

BLOCK = 4000
NBLK = CAPACITY // BLOCK


def _body(q_ref, k_ref, v_ref, recall_ref, best_ref, acc_ref, m_ref, l_ref):
    i = pl.program_id(0)

    @pl.when(i == 0)
    def _init():
        m_ref[0, 0] = -jnp.inf
        l_ref[0, 0] = 0.0
        acc_ref[...] = jnp.zeros_like(acc_ref)

    q = q_ref[0, :]
    qn = q / jnp.maximum(jnp.sqrt(jnp.sum(q * q)), EPS)

    s = jax.lax.dot_general(
        qn[None, :], k_ref[...],
        dimension_numbers=(((1,), (1,)), ((), ())),
        preferred_element_type=jnp.float32,
    )

    m_prev = m_ref[0, 0]
    m_new = jnp.maximum(m_prev, jnp.max(s))
    c = jnp.exp(SCALE * (m_prev - m_new))
    p = jnp.exp(SCALE * (s - m_new))
    l_ref[0, 0] = l_ref[0, 0] * c + jnp.sum(p)
    pv = jax.lax.dot_general(
        p, v_ref[...],
        dimension_numbers=(((1,), (0,)), ((), ())),
        preferred_element_type=jnp.float32,
    )
    acc_ref[...] = acc_ref[...] * c + pv
    m_ref[0, 0] = m_new

    @pl.when(i == NBLK - 1)
    def _fin():
        best = m_ref[0, 0]
        r = acc_ref[...] / l_ref[0, 0]
        recall_ref[...] = jnp.where(best >= THRESHOLD, r, jnp.zeros_like(r))
        best_ref[...] = jnp.full((1, 1), best, dtype=jnp.float32)


@jax.jit
def kernel(query_pattern, keys, values):
    q2 = query_pattern.reshape(1, INPUT_DIM)
    recall, best = pl.pallas_call(
        _body,
        grid=(NBLK,),
        in_specs=[
            pl.BlockSpec((1, INPUT_DIM), lambda i: (0, 0)),
            pl.BlockSpec((BLOCK, INPUT_DIM), lambda i: (i, 0)),
            pl.BlockSpec((BLOCK, VALUE_DIM), lambda i: (i, 0)),
        ],
        out_specs=[
            pl.BlockSpec((1, VALUE_DIM), lambda i: (0, 0)),
            pl.BlockSpec((1, 1), lambda i: (0, 0)),
        ],
        out_shape=[
            jax.ShapeDtypeStruct((1, VALUE_DIM), jnp.float32),
            jax.ShapeDtypeStruct((1, 1), jnp.float32),
        ],
        scratch_shapes=[
            pltpu.VMEM((1, VALUE_DIM), jnp.float32),
            pltpu.SMEM((1, 1), jnp.float32),
            pltpu.SMEM((1, 1), jnp.float32),
        ],
        compiler_params=pltpu.CompilerParams(
            dimension_semantics=("arbitrary",),
        ),
    )(q2, keys, values)
    return recall[0], best[0, 0]
